# baseline (speedup 1.0000x reference)
import math

import jax
import jax.numpy as jnp
import numpy as np
from jax.experimental import pallas as pl
from jax.experimental.pallas import tpu as pltpu

_N_SCALES = 3
_N_FREQ = 16
_N_EXPERTS = 32
_N_LAYERS = 4
_BT = 128


def _bands_const():
    bands = []
    for i in range(_N_SCALES):
        scale = 10.0 ** (i * 0.5)
        bands.append(np.logspace(math.log10(scale * 0.1), math.log10(scale * 10.0), _N_FREQ))
    return jnp.asarray(np.stack(bands), dtype=jnp.float32)


def _compute_h(Wh_ref, ef_ref, ea_ref, wf_ref, wp_ref, wa_ref, bands_ref, T):
    tcol = jax.lax.broadcasted_iota(jnp.int32, (T, 1), 0).astype(jnp.float32)

    feats = []
    for s in range(_N_SCALES):
        args = tcol * bands_ref[s, :][None, :] * (2.0 * math.pi / T)
        feats.append(jnp.sin(args))
        feats.append(jnp.cos(args))
    h = jnp.concatenate(feats, axis=-1)

    t_norm = tcol / T * 2.0 * math.pi

    for l in range(_N_LAYERS):
        wbar = jnp.sum(Wh_ref[l], axis=0)
        s_val = jnp.abs(jnp.sum(h * wbar[None, :], axis=1, keepdims=True))
        s_val = s_val - _N_EXPERTS * jnp.floor(s_val * (1.0 / _N_EXPERTS))
        idx = s_val.astype(jnp.int32)

        onehot = (idx == jax.lax.broadcasted_iota(jnp.int32, (T, _N_EXPERTS), 1)
                  ).astype(jnp.float32)
        F = jnp.dot(onehot, ef_ref[l], preferred_element_type=jnp.float32)
        A = jnp.dot(onehot, ea_ref[l], preferred_element_type=jnp.float32)
        mod = jnp.sum(A * jnp.sin(F * t_norm), axis=1, keepdims=True)
        hr = h * (1.0 + 0.1 * mod)

        waves = jnp.sin(tcol * wf_ref[l, :][None, :] + wp_ref[l, :][None, :])
        interf = jnp.dot(waves, wa_ref[l], preferred_element_type=jnp.float32)
        h = h + 0.5 * (hr + interf)
    return h


def _fused_body(Wh_ref, ef_ref, ea_ref, wf_ref, wp_ref, wa_ref, bands_ref,
                w_ref, b_ref, o_ref, h_scr):
    i = pl.program_id(0)
    T = h_scr.shape[0]

    @pl.when(i == 0)
    def _():
        h_scr[...] = _compute_h(Wh_ref, ef_ref, ea_ref, wf_ref, wp_ref,
                                wa_ref, bands_ref, T)

    h_blk = h_scr[pl.ds(i * _BT, _BT), :]
    logits = jax.lax.dot_general(
        h_blk, w_ref[...],
        dimension_numbers=(((1,), (1,)), ((), ())),
        preferred_element_type=jnp.float32,
    ) + b_ref[...]
    o_ref[...] = jnp.broadcast_to(logits[None], o_ref.shape)


def kernel(x, W_hash, expert_freqs, expert_amps, wave_freqs, wave_phases,
           wave_amps, W_out, b_out):
    B, T = x.shape
    V, D = W_out.shape

    full = lambda shape: pl.BlockSpec(shape, lambda i: (0,) * len(shape))
    out = pl.pallas_call(
        _fused_body,
        grid=(T // _BT,),
        in_specs=[
            full(W_hash.shape),
            full(expert_freqs.shape),
            full(expert_amps.shape),
            full(wave_freqs.shape),
            full(wave_phases.shape),
            full(wave_amps.shape),
            full((_N_SCALES, _N_FREQ)),
            full((V, D)),
            full((1, V)),
        ],
        out_specs=pl.BlockSpec((B, _BT, V), lambda i: (0, i, 0)),
        out_shape=jax.ShapeDtypeStruct((B, T, V), jnp.float32),
        scratch_shapes=[pltpu.VMEM((T, D), jnp.float32)],
    )(W_hash, expert_freqs, expert_amps, wave_freqs, wave_phases, wave_amps,
      _bands_const(), W_out, b_out.reshape(1, V))
    return out


# speedup vs baseline: 1.2027x; 1.2027x over previous
import math

import jax
import jax.numpy as jnp
import numpy as np
from jax.experimental import pallas as pl
from jax.experimental.pallas import tpu as pltpu

_N_SCALES = 3
_N_FREQ = 16
_N_EXPERTS = 32
_N_LAYERS = 4


def _bands_const():
    bands = []
    for i in range(_N_SCALES):
        scale = 10.0 ** (i * 0.5)
        bands.append(np.logspace(math.log10(scale * 0.1), math.log10(scale * 10.0), _N_FREQ))
    return jnp.asarray(np.stack(bands), dtype=jnp.float32)


def _route_body(Wh_ref, ef_ref, ea_ref, wf_ref, wp_ref, wa_ref, bands_ref, h_ref):
    T = h_ref.shape[0]
    tcol = jax.lax.broadcasted_iota(jnp.int32, (T, 1), 0).astype(jnp.float32)

    feats = []
    for s in range(_N_SCALES):
        args = tcol * bands_ref[s, :][None, :] * (2.0 * math.pi / T)
        feats.append(jnp.sin(args))
        feats.append(jnp.cos(args))
    h = jnp.concatenate(feats, axis=-1)

    t_norm = tcol / T * 2.0 * math.pi

    for l in range(_N_LAYERS):
        wbar = jnp.sum(Wh_ref[l], axis=0)
        s_val = jnp.abs(jnp.sum(h * wbar[None, :], axis=1, keepdims=True))
        s_val = s_val - _N_EXPERTS * jnp.floor(s_val * (1.0 / _N_EXPERTS))
        idx = s_val.astype(jnp.int32)

        onehot = (idx == jax.lax.broadcasted_iota(jnp.int32, (T, _N_EXPERTS), 1)
                  ).astype(jnp.float32)
        F = jnp.dot(onehot, ef_ref[l], preferred_element_type=jnp.float32)
        A = jnp.dot(onehot, ea_ref[l], preferred_element_type=jnp.float32)
        mod = jnp.sum(A * jnp.sin(F * t_norm), axis=1, keepdims=True)
        hr = h * (1.0 + 0.1 * mod)

        waves = jnp.sin(tcol * wf_ref[l, :][None, :] + wp_ref[l, :][None, :])
        interf = jnp.dot(waves, wa_ref[l], preferred_element_type=jnp.float32)
        h = h + 0.5 * (hr + interf)

    h_ref[...] = h


def _proj_body(h_ref, w_ref, b_ref, o_ref):
    logits = jax.lax.dot_general(
        h_ref[...], w_ref[...],
        dimension_numbers=(((1,), (1,)), ((), ())),
        preferred_element_type=jnp.float32,
    ) + b_ref[...]
    o_ref[...] = jnp.broadcast_to(logits[None], o_ref.shape)


def kernel(x, W_hash, expert_freqs, expert_amps, wave_freqs, wave_phases,
           wave_amps, W_out, b_out):
    B, T = x.shape
    V, D = W_out.shape

    h = pl.pallas_call(
        _route_body,
        out_shape=jax.ShapeDtypeStruct((T, D), jnp.float32),
    )(W_hash, expert_freqs, expert_amps, wave_freqs, wave_phases, wave_amps,
      _bands_const())

    bt = 128
    out = pl.pallas_call(
        _proj_body,
        grid=(T // bt,),
        in_specs=[
            pl.BlockSpec((bt, D), lambda i: (i, 0)),
            pl.BlockSpec((V, D), lambda i: (0, 0)),
            pl.BlockSpec((1, V), lambda i: (0, 0)),
        ],
        out_specs=pl.BlockSpec((B, bt, V), lambda i: (0, i, 0)),
        out_shape=jax.ShapeDtypeStruct((B, T, V), jnp.float32),
        compiler_params=pltpu.CompilerParams(
            dimension_semantics=("parallel",)),
    )(h, W_out, b_out.reshape(1, V))
    return out


# speedup vs baseline: 1.2862x; 1.0694x over previous
import functools
import math

import jax
import jax.numpy as jnp
import numpy as np
from jax import lax
from jax.experimental import pallas as pl
from jax.experimental.pallas import tpu as pltpu
from jax.experimental.pallas import tpu_sc as plsc

_N_SCALES = 3
_N_FREQ = 16
_N_EXPERTS = 32
_N_LAYERS = 4
_N_WAVES = 16
_LANES = 16

_TWO_PI = np.float32(2.0 * math.pi)


def _bands_const():
    bands = []
    for i in range(_N_SCALES):
        scale = 10.0 ** (i * 0.5)
        bands.append(np.logspace(math.log10(scale * 0.1), math.log10(scale * 10.0), _N_FREQ))
    return jnp.asarray(np.stack(bands), dtype=jnp.float32)


def _sin_poly_coeffs():
    y = np.linspace(-np.pi, np.pi, 20001)
    y2 = y * y
    A = np.stack([y2**k for k in range(7)], axis=-1)
    c, *_ = np.linalg.lstsq(A, np.where(y == 0, 1.0, np.sin(y) / np.where(y == 0, 1.0, y)), rcond=None)
    return [np.float32(v) for v in c]


_SIN_C = _sin_poly_coeffs()


def _sc_sin(x):
    r = lax.rem(x, _TWO_PI)
    y = jnp.where(r > np.float32(math.pi), r - _TWO_PI, r)
    y2 = y * y
    p = _SIN_C[6]
    for k in range(5, -1, -1):
        p = p * y2 + _SIN_C[k]
    return y * p


def _pre_body(Wh_ref, wf_ref, wp_ref, wa_ref, bands_ref,
              hT_ref, ci_ref, wb_ref):
    D, T = hT_ref.shape
    trow = lax.broadcasted_iota(jnp.int32, (1, T), 1).astype(jnp.float32)

    feats = []
    for s in range(_N_SCALES):
        args = trow * bands_ref[s, :][:, None] * (2.0 * math.pi / T)
        feats.append(jnp.sin(args))
        feats.append(jnp.cos(args))
    hT_ref[...] = jnp.concatenate(feats, axis=0)

    for l in range(_N_LAYERS):
        wavesT = jnp.sin(trow * wf_ref[l, :][:, None] + wp_ref[l, :][:, None])
        interfT = lax.dot_general(
            wa_ref[l], wavesT, dimension_numbers=(((0,), (0,)), ((), ())),
            preferred_element_type=jnp.float32)
        ci_ref[l] = 0.5 * interfT
        wbar = jnp.sum(Wh_ref[l], axis=0)
        wb_ref[l] = jnp.broadcast_to(wbar[:, None], (D, _LANES))


def _sc_route_body(hT0_hbm, ci_hbm, wb_hbm, ef_hbm, ea_hbm, out_hbm,
                   h_v, ci_v, wb_v, ef_v, ea_v, ncores, nworkers):
    T = hT0_hbm.shape[1]
    cpt = T // nworkers
    wid = lax.axis_index("s") * ncores + lax.axis_index("c")
    base = wid * cpt

    @pl.when(wid < nworkers)
    def _():
        pltpu.sync_copy(hT0_hbm.at[:, pl.ds(base, cpt)], h_v)
        for l in range(_N_LAYERS):
            pltpu.sync_copy(ci_hbm.at[l, :, pl.ds(base, cpt)], ci_v.at[l])
        pltpu.sync_copy(wb_hbm, wb_v)
        pltpu.sync_copy(ef_hbm, ef_v)
        pltpu.sync_copy(ea_hbm, ea_v)

        D = 2 * _N_SCALES * _N_FREQ
        ngroups = cpt // _LANES
        lane = lax.iota(jnp.int32, _LANES)

        for l in range(_N_LAYERS):
            def dbody(d, accs):
                w = wb_v[l, d]
                return tuple(accs[g] + h_v[d, pl.ds(g * _LANES, _LANES)] * w
                             for g in range(ngroups))

            accs = lax.fori_loop(0, D, dbody,
                                 tuple(jnp.zeros((_LANES,), jnp.float32)
                                       for _ in range(ngroups)))

            alphas = []
            for g in range(ngroups):
                s_val = lax.rem(jnp.abs(accs[g]), np.float32(_N_EXPERTS))
                idx = s_val.astype(jnp.int32)
                t_ids = base + g * _LANES + lane
                tn = t_ids.astype(jnp.float32) / 2048.0 * 2.0 * math.pi
                mod = jnp.zeros((_LANES,), jnp.float32)
                for j in range(8):
                    fidx = (l * 8 + j) * _N_EXPERTS + idx
                    F = plsc.load_gather(ef_v, [fidx])
                    A = plsc.load_gather(ea_v, [fidx])
                    mod = mod + A * _sc_sin(F * tn)
                alphas.append(1.5 + 0.05 * mod)

            def ubody(d, _):
                for g in range(ngroups):
                    sl = pl.ds(g * _LANES, _LANES)
                    h_v[d, sl] = h_v[d, sl] * alphas[g] + ci_v[l, d, sl]
                return 0

            lax.fori_loop(0, D, ubody, 0)

        pltpu.sync_copy(h_v, out_hbm.at[:, pl.ds(base, cpt)])


def _proj_body(hT_ref, w_ref, b_ref, o_ref):
    logits = lax.dot_general(
        hT_ref[...], w_ref[...],
        dimension_numbers=(((0,), (1,)), ((), ())),
        preferred_element_type=jnp.float32,
    ) + b_ref[...]
    o_ref[...] = jnp.broadcast_to(logits[None], o_ref.shape)


def kernel(x, W_hash, expert_freqs, expert_amps, wave_freqs, wave_phases,
           wave_amps, W_out, b_out):
    B, T = x.shape
    V, D = W_out.shape
    L = W_hash.shape[0]

    hT0, cinterfT, wbarB = pl.pallas_call(
        _pre_body,
        out_shape=(
            jax.ShapeDtypeStruct((D, T), jnp.float32),
            jax.ShapeDtypeStruct((L, D, T), jnp.float32),
            jax.ShapeDtypeStruct((L, D, _LANES), jnp.float32),
        ),
    )(W_hash, wave_freqs, wave_phases, wave_amps, _bands_const())

    efT = jnp.transpose(expert_freqs, (0, 2, 1)).reshape(-1)
    eaT = jnp.transpose(expert_amps, (0, 2, 1)).reshape(-1)

    info = plsc.get_sparse_core_info()
    nc, ns = info.num_cores, info.num_subcores
    nworkers = 16
    cpt = T // nworkers
    mesh = plsc.VectorSubcoreMesh(core_axis_name="c", subcore_axis_name="s")
    hT = pl.kernel(
        functools.partial(_sc_route_body, ncores=nc, nworkers=nworkers),
        out_type=jax.ShapeDtypeStruct((D, T), jnp.float32),
        mesh=mesh,
        compiler_params=pltpu.CompilerParams(needs_layout_passes=False),
        scratch_types=[
            pltpu.VMEM((D, cpt), jnp.float32),
            pltpu.VMEM((L, D, cpt), jnp.float32),
            pltpu.VMEM((L, D, _LANES), jnp.float32),
            pltpu.VMEM((L * 8 * _N_EXPERTS,), jnp.float32),
            pltpu.VMEM((L * 8 * _N_EXPERTS,), jnp.float32),
        ],
    )(hT0, cinterfT, wbarB, efT, eaT)

    bt = 128
    out = pl.pallas_call(
        _proj_body,
        grid=(T // bt,),
        in_specs=[
            pl.BlockSpec((D, bt), lambda i: (0, i)),
            pl.BlockSpec((V, D), lambda i: (0, 0)),
            pl.BlockSpec((1, V), lambda i: (0, 0)),
        ],
        out_specs=pl.BlockSpec((B, bt, V), lambda i: (0, i, 0)),
        out_shape=jax.ShapeDtypeStruct((B, T, V), jnp.float32),
        compiler_params=pltpu.CompilerParams(
            dimension_semantics=("parallel",)),
    )(hT, W_out, b_out.reshape(1, V))
    return out
